# Initial kernel scaffold; baseline (speedup 1.0000x reference)
import jax
import jax.numpy as jnp
from jax.experimental import pallas as pl


def kernel(hidden_states, cos, sin, Wq, Wk, Wv, Wo):
    raise NotImplementedError("write your pallas kernel here")


# speedup vs baseline: 1.1353x; 1.1353x over previous
import functools

import jax
import jax.numpy as jnp
from jax.experimental import pallas as pl
from jax.experimental.pallas import tpu as pltpu

_B, _S, _D = 1, 2048, 768
_H, _KVH, _HD = 12, 4, 64
_REP = _H // _KVH
_SCALE = _HD ** -0.5
_QKV = (_H + 2 * _KVH) * _HD
_ROPE_W = (_H + _KVH) * _HD
_BQ = 256
_NEG = -1e9


def _qkv_rope_kernel(hid_ref, w_ref, cos_ref, sin_ref, q_ref, k_ref, v_ref):
    qkv = jnp.dot(hid_ref[...], w_ref[...], preferred_element_type=jnp.float32)
    cos = cos_ref[...]
    sin = sin_ref[...]
    nrep = _ROPE_W // _HD
    cos_t = jnp.concatenate([cos] * nrep, axis=-1)
    sin_t = jnp.concatenate([sin] * nrep, axis=-1)
    qk = qkv[:, :_ROPE_W]
    lane = jax.lax.broadcasted_iota(jnp.int32, (_BQ, _ROPE_W), 1)
    first_half = (lane % _HD) < (_HD // 2)
    rot = jnp.where(first_half, -pltpu.roll(qk, _ROPE_W - 32, 1),
                    pltpu.roll(qk, 32, 1))
    qk = qk * cos_t + rot * sin_t
    for h in range(_H):
        q_ref[h] = qk[:, h * _HD:(h + 1) * _HD]
    for g in range(_KVH):
        k_ref[g] = qk[:, (_H + g) * _HD:(_H + g + 1) * _HD]
        v_ref[g] = qkv[:, (_H + _KVH + g) * _HD:(_H + _KVH + g + 1) * _HD]


def _attn_kernel(q_ref, k_ref, v_ref, wo_ref, out_ref):
    qb = pl.program_id(0)
    h = pl.program_id(1)
    q = q_ref[0]
    k = k_ref[0]
    v = v_ref[0]
    s = jax.lax.dot_general(q, k, (((1,), (1,)), ((), ())),
                            preferred_element_type=jnp.float32)
    row = qb * _BQ + jax.lax.broadcasted_iota(jnp.int32, (_BQ, _S), 0)
    col = jax.lax.broadcasted_iota(jnp.int32, (_BQ, _S), 1)
    s = jnp.where(col <= row, s, _NEG)
    m = jnp.max(s, axis=-1, keepdims=True)
    e = jnp.exp(s - m)
    l = jnp.sum(e, axis=-1, keepdims=True)
    o = jnp.dot(e, v, preferred_element_type=jnp.float32) / l
    acc = jnp.dot(o, wo_ref[0], preferred_element_type=jnp.float32)

    @pl.when(h == 0)
    def _():
        out_ref[...] = acc

    @pl.when(h != 0)
    def _():
        out_ref[...] += acc


@functools.partial(jax.jit, static_argnames=())
def kernel(hidden_states, cos, sin, Wq, Wk, Wv, Wo):
    hid = hidden_states.reshape(_S, _D)
    cos2 = cos.reshape(_S, _HD)
    sin2 = sin.reshape(_S, _HD)
    w_qkv = jnp.concatenate([Wq * _SCALE, Wk, Wv], axis=1)
    wo3 = Wo.reshape(_H, _HD, _D)

    nq = _S // _BQ
    q, k, v = pl.pallas_call(
        _qkv_rope_kernel,
        grid=(nq,),
        in_specs=[
            pl.BlockSpec((_BQ, _D), lambda i: (i, 0)),
            pl.BlockSpec((_D, _QKV), lambda i: (0, 0)),
            pl.BlockSpec((_BQ, _HD), lambda i: (i, 0)),
            pl.BlockSpec((_BQ, _HD), lambda i: (i, 0)),
        ],
        out_specs=[
            pl.BlockSpec((_H, _BQ, _HD), lambda i: (0, i, 0)),
            pl.BlockSpec((_KVH, _BQ, _HD), lambda i: (0, i, 0)),
            pl.BlockSpec((_KVH, _BQ, _HD), lambda i: (0, i, 0)),
        ],
        out_shape=[
            jax.ShapeDtypeStruct((_H, _S, _HD), jnp.float32),
            jax.ShapeDtypeStruct((_KVH, _S, _HD), jnp.float32),
            jax.ShapeDtypeStruct((_KVH, _S, _HD), jnp.float32),
        ],
    )(hid, w_qkv, cos2, sin2)

    out = pl.pallas_call(
        _attn_kernel,
        grid=(nq, _H),
        in_specs=[
            pl.BlockSpec((1, _BQ, _HD), lambda i, j: (j, i, 0)),
            pl.BlockSpec((1, _S, _HD), lambda i, j: (j // _REP, 0, 0)),
            pl.BlockSpec((1, _S, _HD), lambda i, j: (j // _REP, 0, 0)),
            pl.BlockSpec((1, _HD, _D), lambda i, j: (j, 0, 0)),
        ],
        out_specs=pl.BlockSpec((_BQ, _D), lambda i, j: (i, 0)),
        out_shape=jax.ShapeDtypeStruct((_S, _D), jnp.float32),
    )(q, k, v, wo3)

    return out.reshape(_B, _S, _D)


# speedup vs baseline: 1.3589x; 1.1970x over previous
import functools

import jax
import jax.numpy as jnp
from jax.experimental import pallas as pl
from jax.experimental.pallas import tpu as pltpu

_B, _S, _D = 1, 2048, 768
_H, _KVH, _HD = 12, 4, 64
_REP = _H // _KVH
_SCALE = _HD ** -0.5
_QKV = (_H + 2 * _KVH) * _HD
_ROPE_W = (_H + _KVH) * _HD
_BQ = 256
_NEG = -1e9


def _qkv_rope_kernel(hid_ref, w_ref, cos_ref, sin_ref, q_ref, k_ref, v_ref):
    qkv = jnp.dot(hid_ref[...], w_ref[...], preferred_element_type=jnp.float32)
    cos = cos_ref[...]
    sin = sin_ref[...]
    nrep = _ROPE_W // _HD
    cos_t = jnp.concatenate([cos] * nrep, axis=-1)
    sin_t = jnp.concatenate([sin] * nrep, axis=-1)
    qk = qkv[:, :_ROPE_W]
    lane = jax.lax.broadcasted_iota(jnp.int32, (_BQ, _ROPE_W), 1)
    first_half = (lane % _HD) < (_HD // 2)
    rot = jnp.where(first_half, -pltpu.roll(qk, _ROPE_W - 32, 1),
                    pltpu.roll(qk, 32, 1))
    qk = (qk * cos_t + rot * sin_t).astype(jnp.bfloat16)
    vv = qkv[:, _ROPE_W:].astype(jnp.bfloat16)
    for h in range(_H):
        q_ref[h] = qk[:, h * _HD:(h + 1) * _HD]
    for g in range(_KVH):
        k_ref[g] = qk[:, (_H + g) * _HD:(_H + g + 1) * _HD]
        v_ref[g] = vv[:, g * _HD:(g + 1) * _HD]


def _attn_kernel(q_ref, k_ref, v_ref, wo_ref, out_ref):
    qb = pl.program_id(0)
    h = pl.program_id(1)
    q = q_ref[0]
    k = k_ref[0]
    v = v_ref[0]
    s = jax.lax.dot_general(q, k, (((1,), (1,)), ((), ())),
                            preferred_element_type=jnp.float32)
    row = qb * _BQ + jax.lax.broadcasted_iota(jnp.int32, (_BQ, _S), 0)
    col = jax.lax.broadcasted_iota(jnp.int32, (_BQ, _S), 1)
    s = jnp.where(col <= row, s, _NEG)
    m = jnp.max(s, axis=-1, keepdims=True)
    e = jnp.exp(s - m)
    l = jnp.sum(e, axis=-1, keepdims=True)
    o = jnp.dot(e.astype(jnp.bfloat16), v,
                preferred_element_type=jnp.float32) / l
    acc = jnp.dot(o.astype(jnp.bfloat16), wo_ref[0],
                  preferred_element_type=jnp.float32)

    @pl.when(h == 0)
    def _():
        out_ref[...] = acc

    @pl.when(h != 0)
    def _():
        out_ref[...] += acc


@functools.partial(jax.jit, static_argnames=())
def kernel(hidden_states, cos, sin, Wq, Wk, Wv, Wo):
    hid = hidden_states.reshape(_S, _D).astype(jnp.bfloat16)
    cos2 = cos.reshape(_S, _HD)
    sin2 = sin.reshape(_S, _HD)
    w_qkv = jnp.concatenate([Wq * _SCALE, Wk, Wv],
                            axis=1).astype(jnp.bfloat16)
    wo3 = Wo.reshape(_H, _HD, _D).astype(jnp.bfloat16)

    nq = _S // _BQ
    q, k, v = pl.pallas_call(
        _qkv_rope_kernel,
        grid=(nq,),
        in_specs=[
            pl.BlockSpec((_BQ, _D), lambda i: (i, 0)),
            pl.BlockSpec((_D, _QKV), lambda i: (0, 0)),
            pl.BlockSpec((_BQ, _HD), lambda i: (i, 0)),
            pl.BlockSpec((_BQ, _HD), lambda i: (i, 0)),
        ],
        out_specs=[
            pl.BlockSpec((_H, _BQ, _HD), lambda i: (0, i, 0)),
            pl.BlockSpec((_KVH, _BQ, _HD), lambda i: (0, i, 0)),
            pl.BlockSpec((_KVH, _BQ, _HD), lambda i: (0, i, 0)),
        ],
        out_shape=[
            jax.ShapeDtypeStruct((_H, _S, _HD), jnp.bfloat16),
            jax.ShapeDtypeStruct((_KVH, _S, _HD), jnp.bfloat16),
            jax.ShapeDtypeStruct((_KVH, _S, _HD), jnp.bfloat16),
        ],
    )(hid, w_qkv, cos2, sin2)

    out = pl.pallas_call(
        _attn_kernel,
        grid=(nq, _H),
        in_specs=[
            pl.BlockSpec((1, _BQ, _HD), lambda i, j: (j, i, 0)),
            pl.BlockSpec((1, _S, _HD), lambda i, j: (j // _REP, 0, 0)),
            pl.BlockSpec((1, _S, _HD), lambda i, j: (j // _REP, 0, 0)),
            pl.BlockSpec((1, _HD, _D), lambda i, j: (j, 0, 0)),
        ],
        out_specs=pl.BlockSpec((_BQ, _D), lambda i, j: (i, 0)),
        out_shape=jax.ShapeDtypeStruct((_S, _D), jnp.float32),
    )(q, k, v, wo3)

    return out.reshape(_B, _S, _D)
